# baseline (speedup 1.0000x reference)
import functools

import jax
import jax.numpy as jnp
from jax import lax
from jax.experimental import pallas as pl
from jax.experimental.pallas import tpu as pltpu
from jax.experimental.pallas import tpu_sc as plsc

B, H, D = 4096, 200, 32
VOCAB = 1000
TV = VOCAB * D

NC = 2
NS = 16
NW = NC * NS
B_PER_W = B // NW
IDX_BLK = 32


@functools.partial(
    pl.kernel,
    out_type=jax.ShapeDtypeStruct((B, H, D), jnp.float32),
    mesh=plsc.VectorSubcoreMesh(
        core_axis_name="c", subcore_axis_name="s", num_cores=NC, num_subcores=NS
    ),
    scratch_types=[
        pltpu.VMEM((TV,), jnp.float32),
        pltpu.VMEM((IDX_BLK, H), jnp.int32),
        pltpu.VMEM((H, D), jnp.float32),
        pltpu.VMEM((H, D), jnp.float32),
        pltpu.SemaphoreType.DMA,
        pltpu.SemaphoreType.DMA,
    ],
    compiler_params=pltpu.CompilerParams(needs_layout_passes=False),
)
def _gather_kernel(table_hbm, items_hbm, out_hbm, table_v, idx_v,
                   rows_a, rows_b, sem_a, sem_b):
    wid = lax.axis_index("s") * NC + lax.axis_index("c")
    b0 = wid * B_PER_W
    pltpu.sync_copy(table_hbm, table_v)

    iota0 = lax.iota(jnp.int32, 16)
    lane_consts = [jnp.full((16,), l, jnp.int32) for l in range(D)]

    def do_group(buf, rvec, idxv, mask):
        idx32 = idxv * D
        for l in range(D):
            g = plsc.load_gather(table_v, [idx32 + l])
            plsc.store_scatter(buf, [rvec, lane_consts[l]], g, mask=mask)

    def fill(buf, bb):
        def group_body(g, _):
            r0 = g * 16
            idxv = idx_v[bb, pl.ds(r0, 16)]
            do_group(buf, r0 + iota0, idxv, None)
            return 0
        lax.fori_loop(0, H // 16, group_body, 0)
        idxv = idx_v[bb, pl.ds(H - 16, 16)]
        do_group(buf, (H - 16) + iota0, idxv, iota0 >= 8)

    def blk_body(k, _):
        pltpu.sync_copy(items_hbm.at[pl.ds(b0 + k * IDX_BLK, IDX_BLK)], idx_v)

        def pair_body(p, _):
            for buf, sem, off in ((rows_a, sem_a, 0), (rows_b, sem_b, 1)):
                bb = 2 * p + off
                b = b0 + k * IDX_BLK + bb

                @pl.when(jnp.logical_or(p > 0, k > 0))
                def _drain():
                    pltpu.make_async_copy(buf, out_hbm.at[b], sem).wait()

                fill(buf, bb)
                pltpu.async_copy(buf, out_hbm.at[b], sem)
            return 0

        lax.fori_loop(0, IDX_BLK // 2, pair_body, 0)
        return 0

    lax.fori_loop(0, B_PER_W // IDX_BLK, blk_body, 0)

    last = b0 + B_PER_W - 1
    pltpu.make_async_copy(rows_a, out_hbm.at[last - 1], sem_a).wait()
    pltpu.make_async_copy(rows_b, out_hbm.at[last], sem_b).wait()


def kernel(items, timesteps, item_embedding_table):
    del timesteps
    table_flat = item_embedding_table.reshape(-1)
    return _gather_kernel(table_flat, items.astype(jnp.int32))


# speedup vs baseline: 2.4721x; 2.4721x over previous
import functools

import jax
import jax.numpy as jnp
from jax import lax
from jax.experimental import pallas as pl
from jax.experimental.pallas import tpu as pltpu
from jax.experimental.pallas import tpu_sc as plsc

B, H, D = 4096, 200, 32
VOCAB = 1000
TV = VOCAB * D

NC = 2
NS = 16
NW = NC * NS
B_PER_W = B // NW
IDX_BLK = 32


@functools.partial(
    pl.kernel,
    out_type=jax.ShapeDtypeStruct((B, H, D), jnp.float32),
    mesh=plsc.VectorSubcoreMesh(
        core_axis_name="c", subcore_axis_name="s", num_cores=NC, num_subcores=NS
    ),
    scratch_types=[
        pltpu.VMEM((TV,), jnp.float32),
        pltpu.VMEM((IDX_BLK, H), jnp.int32),
        pltpu.VMEM((H, D), jnp.float32),
        pltpu.VMEM((H, D), jnp.float32),
        pltpu.SemaphoreType.DMA,
        pltpu.SemaphoreType.DMA,
    ],
    compiler_params=pltpu.CompilerParams(needs_layout_passes=False),
)
def _gather_kernel(table_hbm, items_hbm, out_hbm, table_v, idx_v,
                   rows_a, rows_b, sem_a, sem_b):
    wid = lax.axis_index("s") * NC + lax.axis_index("c")
    b0 = wid * B_PER_W
    pltpu.sync_copy(table_hbm, table_v)

    iota0 = lax.iota(jnp.int32, 16)
    lane_consts = [jnp.full((16,), l, jnp.int32) for l in range(16)]

    def do_group(buf, r0, idxv, lanes):
        idx32 = idxv * D
        for dr in lanes:
            s = idx32.at[lane_consts[dr]].get(mode="promise_in_bounds")
            a0 = s + iota0
            r = r0 + dr
            buf[r, pl.ds(0, 16)] = plsc.load_gather(table_v, [a0])
            buf[r, pl.ds(16, 16)] = plsc.load_gather(table_v, [a0 + 16])

    def fill(buf, bb):
        def group_body(g, _):
            r0 = g * 16
            idxv = idx_v[bb, pl.ds(r0, 16)]
            do_group(buf, r0, idxv, range(16))
            return 0
        lax.fori_loop(0, H // 16, group_body, 0)
        idxv = idx_v[bb, pl.ds(H - 16, 16)]
        do_group(buf, H - 16, idxv, range(8, 16))

    def blk_body(k, _):
        pltpu.sync_copy(items_hbm.at[pl.ds(b0 + k * IDX_BLK, IDX_BLK)], idx_v)

        def pair_body(p, _):
            for buf, sem, off in ((rows_a, sem_a, 0), (rows_b, sem_b, 1)):
                bb = 2 * p + off
                b = b0 + k * IDX_BLK + bb

                @pl.when(jnp.logical_or(p > 0, k > 0))
                def _drain():
                    pltpu.make_async_copy(buf, out_hbm.at[b], sem).wait()

                fill(buf, bb)
                pltpu.async_copy(buf, out_hbm.at[b], sem)
            return 0

        lax.fori_loop(0, IDX_BLK // 2, pair_body, 0)
        return 0

    lax.fori_loop(0, B_PER_W // IDX_BLK, blk_body, 0)

    last = b0 + B_PER_W - 1
    pltpu.make_async_copy(rows_a, out_hbm.at[last - 1], sem_a).wait()
    pltpu.make_async_copy(rows_b, out_hbm.at[last], sem_b).wait()


def kernel(items, timesteps, item_embedding_table):
    del timesteps
    table_flat = item_embedding_table.reshape(-1)
    return _gather_kernel(table_flat, items.astype(jnp.int32))


# speedup vs baseline: 4.0655x; 1.6445x over previous
import functools

import jax
import jax.numpy as jnp
from jax import lax
from jax.experimental import pallas as pl
from jax.experimental.pallas import tpu as pltpu
from jax.experimental.pallas import tpu_sc as plsc

B, H, D = 4096, 200, 32
VOCAB = 1000
TV = VOCAB * D

NC = 2
NS = 16
NW = NC * NS
N_UNITS = H * (D // 8)
U_PER_W = N_UNITS // NW
NBG = B // 16


@functools.partial(
    pl.kernel,
    out_type=jax.ShapeDtypeStruct((H, D, B), jnp.float32),
    mesh=plsc.VectorSubcoreMesh(
        core_axis_name="c", subcore_axis_name="s", num_cores=NC, num_subcores=NS
    ),
    scratch_types=[
        pltpu.VMEM((TV,), jnp.float32),
        pltpu.VMEM((B,), jnp.int32),
        pltpu.VMEM((8, B), jnp.float32),
        pltpu.VMEM((8, B), jnp.float32),
        pltpu.SemaphoreType.DMA,
    ],
    compiler_params=pltpu.CompilerParams(needs_layout_passes=False),
)
def _gather_kernel(table_hbm, items_hbm, out_hbm, table_v, idx_v,
                   buf_a, buf_b, sem):
    wid = lax.axis_index("s") * NC + lax.axis_index("c")
    u0 = wid * U_PER_W
    pltpu.sync_copy(table_hbm, table_v)

    def fill(buf, d0):
        consts = [jnp.full((16,), (d0 + dd) * VOCAB, jnp.int32)
                  for dd in range(8)]

        def group(g, _):
            j0 = g * 16
            idx16 = idx_v[pl.ds(j0, 16)]
            for dd in range(8):
                buf[dd, pl.ds(j0, 16)] = plsc.load_gather(
                    table_v, [idx16 + consts[dd]])
            return 0

        lax.fori_loop(0, NBG, group, 0)

    def unit_body(i, _):
        u = u0 + i
        h = u // (D // 8)
        d0 = (u % (D // 8)) * 8
        pltpu.sync_copy(items_hbm.at[pl.ds(h * B, B)], idx_v)
        dst = out_hbm.at[h, pl.ds(d0, 8)]

        def run(buf):
            @pl.when(i >= 2)
            def _drain():
                pltpu.make_async_copy(buf, dst, sem).wait()

            fill(buf, d0)
            pltpu.async_copy(buf, dst, sem)

        @pl.when(i % 2 == 0)
        def _even():
            run(buf_a)

        @pl.when(i % 2 == 1)
        def _odd():
            run(buf_b)

        return 0

    lax.fori_loop(0, U_PER_W, unit_body, 0)

    dst0 = out_hbm.at[0, pl.ds(0, 8)]
    pltpu.make_async_copy(buf_a, dst0, sem).wait()
    pltpu.make_async_copy(buf_b, dst0, sem).wait()


def kernel(items, timesteps, item_embedding_table):
    del timesteps
    items_t = items.T.astype(jnp.int32).reshape(-1)
    table_t = item_embedding_table.T.reshape(-1)
    out_t = _gather_kernel(table_t, items_t)
    return jnp.transpose(out_t, (2, 0, 1))


# speedup vs baseline: 4.2796x; 1.0526x over previous
import functools

import jax
import jax.numpy as jnp
from jax import lax
from jax.experimental import pallas as pl
from jax.experimental.pallas import tpu as pltpu
from jax.experimental.pallas import tpu_sc as plsc

B, H, D = 4096, 200, 32
VOCAB = 1000
TV = VOCAB * D

NC = 2
NS = 16
NW = NC * NS
N_UNITS = H * (D // 8)
U_PER_W = N_UNITS // NW
NBG = B // 16


@functools.partial(
    pl.kernel,
    out_type=jax.ShapeDtypeStruct((H, D, B), jnp.float32),
    mesh=plsc.VectorSubcoreMesh(
        core_axis_name="c", subcore_axis_name="s", num_cores=NC, num_subcores=NS
    ),
    scratch_types=[
        pltpu.VMEM((TV,), jnp.float32),
        pltpu.VMEM((B,), jnp.int32),
        pltpu.VMEM((8, B), jnp.float32),
        pltpu.VMEM((8, B), jnp.float32),
        pltpu.SemaphoreType.DMA,
    ],
    compiler_params=pltpu.CompilerParams(needs_layout_passes=False),
)
def _gather_kernel(table_hbm, items_hbm, out_hbm, table_v, idx_v,
                   buf_a, buf_b, sem):
    wid = lax.axis_index("s") * NC + lax.axis_index("c")
    u0 = wid * U_PER_W
    pltpu.sync_copy(table_hbm, table_v)

    def fill(buf, d0):
        consts = [jnp.full((16,), (d0 + dd) * VOCAB, jnp.int32)
                  for dd in range(8)]

        def group4(g4, _):
            for gg in range(4):
                j0 = (g4 * 4 + gg) * 16
                idx16 = idx_v[pl.ds(j0, 16)]
                for dd in range(8):
                    buf[dd, pl.ds(j0, 16)] = plsc.load_gather(
                        table_v, [idx16 + consts[dd]])
            return 0

        lax.fori_loop(0, NBG // 4, group4, 0)

    def do_unit(u, buf, drain):
        h = u // (D // 8)
        d0 = (u % (D // 8)) * 8

        @pl.when(jnp.logical_or(u % (D // 8) == 0, u == u0))
        def _stage():
            pltpu.sync_copy(items_hbm.at[pl.ds(h * B, B)], idx_v)

        dst = out_hbm.at[h, pl.ds(d0, 8)]
        if drain:
            pltpu.make_async_copy(buf, dst, sem).wait()
        fill(buf, d0)
        pltpu.async_copy(buf, dst, sem)

    def pair_body(i, _):
        for off, buf in ((0, buf_a), (1, buf_b)):
            do_unit(u0 + 2 * i + off, buf, True)
        return 0

    do_unit(u0, buf_a, False)
    do_unit(u0 + 1, buf_b, False)
    lax.fori_loop(1, (U_PER_W - 1) // 2, pair_body, 0)
    do_unit(u0 + U_PER_W - 1, buf_a, True)

    dst0 = out_hbm.at[0, pl.ds(0, 8)]
    pltpu.make_async_copy(buf_a, dst0, sem).wait()
    pltpu.make_async_copy(buf_b, dst0, sem).wait()


def kernel(items, timesteps, item_embedding_table):
    del timesteps
    items_t = items.T.astype(jnp.int32).reshape(-1)
    table_t = item_embedding_table.T.reshape(-1)
    out_t = _gather_kernel(table_t, items_t)
    return jnp.transpose(out_t, (2, 0, 1))


# speedup vs baseline: 9.6544x; 2.2559x over previous
import functools

import jax
import jax.numpy as jnp
from jax import lax
from jax.experimental import pallas as pl
from jax.experimental.pallas import tpu as pltpu
from jax.experimental.pallas import tpu_sc as plsc

B, H, D = 4096, 200, 32
VOCAB = 1000
TV = VOCAB * D

NC = 2
NS = 16
NW = NC * NS
N_UNITS = H * (D // 8)
U_PER_W = N_UNITS // NW
NBG = B // 16


@functools.partial(
    pl.kernel,
    out_type=jax.ShapeDtypeStruct((H, D, B), jnp.float32),
    mesh=plsc.VectorSubcoreMesh(
        core_axis_name="c", subcore_axis_name="s", num_cores=NC, num_subcores=NS
    ),
    scratch_types=[
        pltpu.VMEM((TV,), jnp.float32),
        pltpu.VMEM((B,), jnp.int32),
        pltpu.VMEM((8, B), jnp.float32),
        pltpu.VMEM((8, B), jnp.float32),
        pltpu.SemaphoreType.DMA,
    ],
    compiler_params=pltpu.CompilerParams(needs_layout_passes=False),
)
def _gather_kernel(table_hbm, items_hbm, out_hbm, table_v, idx_v,
                   buf_a, buf_b, sem):
    wid = lax.axis_index("s") * NC + lax.axis_index("c")
    u0 = wid * U_PER_W
    pltpu.sync_copy(table_hbm, table_v)

    def fill(buf, d0):
        consts = [jnp.full((16,), (d0 + dd) * VOCAB, jnp.int32)
                  for dd in range(8)]

        def group2(g2, _):
            for gg in range(2):
                j0 = (g2 * 2 + gg) * 16
                idx16 = idx_v[pl.ds(j0, 16)]
                addrs = [idx16 + consts[dd] for dd in range(8)]
                vals = [plsc.load_gather(table_v, [a]) for a in addrs]
                for dd in range(8):
                    buf[dd, pl.ds(j0, 16)] = vals[dd]
            return 0

        lax.fori_loop(0, NBG // 2, group2, 0)

    def do_unit(u, buf, drain):
        h = u // (D // 8)
        d0 = (u % (D // 8)) * 8

        @pl.when(jnp.logical_or(u % (D // 8) == 0, u == u0))
        def _stage():
            pltpu.sync_copy(items_hbm.at[pl.ds(h * B, B)], idx_v)

        dst = out_hbm.at[h, pl.ds(d0, 8)]
        if drain:
            pltpu.make_async_copy(buf, dst, sem).wait()
        fill(buf, d0)
        pltpu.async_copy(buf, dst, sem)

    def pair_body(i, _):
        for off, buf in ((0, buf_a), (1, buf_b)):
            do_unit(u0 + 2 * i + off, buf, True)
        return 0

    do_unit(u0, buf_a, False)
    do_unit(u0 + 1, buf_b, False)
    lax.fori_loop(1, (U_PER_W - 1) // 2, pair_body, 0)
    do_unit(u0 + U_PER_W - 1, buf_a, True)

    dst0 = out_hbm.at[0, pl.ds(0, 8)]
    pltpu.make_async_copy(buf_a, dst0, sem).wait()
    pltpu.make_async_copy(buf_b, dst0, sem).wait()


def kernel(items, timesteps, item_embedding_table):
    del timesteps
    items_t = items.T.astype(jnp.int32).reshape(-1)
    table_t = item_embedding_table.T.reshape(-1)
    out_t = _gather_kernel(table_t, items_t)
    return jnp.transpose(out_t, (2, 0, 1))
